# Initial kernel scaffold; baseline (speedup 1.0000x reference)
import functools

import jax
import jax.numpy as jnp
from jax import lax
from jax.experimental import pallas as pl
from jax.experimental.pallas import tpu as pltpu
from jax.experimental.pallas import tpu_sc as plsc

N = 10000
E = 320000
F = 128
H = 4
O = 32

NC = 2
NS = 16
L = 16

EPT = E // NS
W = 160
NWIN = EPT // W
NPT = N // NS
ROW = 80
HB = 500
GRID = N // HB


def _proj_body(x_ref, w_ref, a_ref, h_ref, s_ref, smax_ref):
    i = pl.program_id(0)
    hb = jnp.dot(x_ref[...], w_ref[...], preferred_element_type=jnp.float32)
    h_ref[...] = hb
    sb = jnp.dot(hb, a_ref[...], preferred_element_type=jnp.float32)
    s_ref[...] = sb
    bm = jnp.max(sb, axis=0, keepdims=True)
    prev = jnp.where(i == 0, jnp.full_like(bm, -jnp.inf), smax_ref[...])
    smax_ref[...] = jnp.maximum(prev, bm)


def _proj(x, w2d, amat):
    return pl.pallas_call(
        _proj_body,
        grid=(GRID,),
        in_specs=[
            pl.BlockSpec((HB, F), lambda i: (i, 0)),
            pl.BlockSpec((F, F), lambda i: (0, 0)),
            pl.BlockSpec((F, 2 * H), lambda i: (0, 0)),
        ],
        out_specs=[
            pl.BlockSpec((HB, F), lambda i: (i, 0)),
            pl.BlockSpec((HB, 2 * H), lambda i: (i, 0)),
            pl.BlockSpec((1, 2 * H), lambda i: (0, 0)),
        ],
        out_shape=[
            jax.ShapeDtypeStruct((N, F), jnp.float32),
            jax.ShapeDtypeStruct((N, 2 * H), jnp.float32),
            jax.ShapeDtypeStruct((1, 2 * H), jnp.float32),
        ],
    )(x, w2d, amat)


def _sc_body(h_hbm, srcr_hbm, dstr_hbm, s1_hbm, s2_hbm, m_hbm, out_hbm,
             src_t, dst_t, s1_t, s2_t, rows64, rows80, aexp_a, aexp_b, mv,
             h2_sh, acc_sh, sem_g, sem_s):
    c = lax.axis_index("c")
    t = lax.axis_index("s")

    pltpu.sync_copy(srcr_hbm.at[t], src_t)
    pltpu.sync_copy(dstr_hbm.at[t], dst_t)
    pltpu.sync_copy(s1_hbm.at[c], s1_t)
    pltpu.sync_copy(s2_hbm.at[c], s2_t)
    pltpu.sync_copy(m_hbm.at[c], mv)
    pltpu.sync_copy(h_hbm.at[pl.ds(t * NPT, NPT), pl.ds(c * 64, 64)],
                    h2_sh.at[pl.ds(t * NPT, NPT)])

    z = jnp.zeros((L,), jnp.float32)
    for r in range(W):
        for k in range(ROW // L):
            rows80[r, pl.ds(k * L, L)] = z
    pltpu.sync_copy(rows80, acc_sh.at[pl.ds(t * NPT, W)])
    pltpu.sync_copy(rows80, acc_sh.at[pl.ds(t * NPT + W, W)])
    pltpu.sync_copy(rows80, acc_sh.at[pl.ds(t * NPT + 2 * W, W)])
    pltpu.sync_copy(rows80.at[pl.ds(0, NPT - 3 * W)],
                    acc_sh.at[pl.ds(t * NPT + 3 * W, NPT - 3 * W)])
    plsc.subcore_barrier()

    m = mv[...]
    ma = m[0]
    mb = m[1]
    col64 = jnp.full((L,), 64, jnp.int32)
    col65 = jnp.full((L,), 65, jnp.int32)
    lane = lax.iota(jnp.int32, L)

    def win_body(w, carry):
        gcp = pltpu.async_copy(h2_sh.at[src_t.at[w]], rows64, sem_g)
        for v in range(W // L):
            sv = src_t[w, pl.ds(v * L, L)]
            dv = dst_t[w, pl.ds(v * L, L)]
            i1 = sv * 2
            i2 = dv * 2
            s1a = plsc.load_gather(s1_t, [i1])
            s1b = plsc.load_gather(s1_t, [i1 + 1])
            s2a = plsc.load_gather(s2_t, [i2])
            s2b = plsc.load_gather(s2_t, [i2 + 1])
            aa = s1a + s2a
            ab = s1b + s2b
            aa = jnp.where(aa > 0, aa, 0.2 * aa) - ma
            ab = jnp.where(ab > 0, ab, 0.2 * ab) - mb
            ea = jnp.exp(aa)
            eb = jnp.exp(ab)
            aexp_a[pl.ds(v * L, L)] = ea
            aexp_b[pl.ds(v * L, L)] = eb
            rowv = lane + (v * L)
            plsc.store_scatter(rows80, [rowv, col64], ea)
            plsc.store_scatter(rows80, [rowv, col65], eb)
        gcp.wait()
        for v in range(W // L):
            ea = aexp_a[pl.ds(v * L, L)]
            eb = aexp_b[pl.ds(v * L, L)]
            for j in range(L):
                e = v * L + j
                sa = ea[j]
                sb = eb[j]
                rows80[e, pl.ds(0, L)] = rows64[e, pl.ds(0, L)] * sa
                rows80[e, pl.ds(L, L)] = rows64[e, pl.ds(L, L)] * sa
                rows80[e, pl.ds(2 * L, L)] = rows64[e, pl.ds(2 * L, L)] * sb
                rows80[e, pl.ds(3 * L, L)] = rows64[e, pl.ds(3 * L, L)] * sb
        pltpu.async_copy(rows80, acc_sh.at[dst_t.at[w]], sem_s, add=True).wait()
        return carry

    lax.fori_loop(0, NWIN, win_body, 0)
    plsc.subcore_barrier()
    pltpu.sync_copy(acc_sh.at[pl.ds(t * NPT, NPT)],
                    out_hbm.at[c, pl.ds(t * NPT, NPT)])


_sc_mesh = plsc.VectorSubcoreMesh(
    core_axis_name="c", subcore_axis_name="s", num_cores=NC, num_subcores=NS)

_sc_call = functools.partial(
    pl.kernel,
    out_type=jax.ShapeDtypeStruct((NC, N, ROW), jnp.float32),
    mesh=_sc_mesh,
    scratch_types=[
        pltpu.VMEM((NWIN, W), jnp.int32),
        pltpu.VMEM((NWIN, W), jnp.int32),
        pltpu.VMEM((2 * N,), jnp.float32),
        pltpu.VMEM((2 * N,), jnp.float32),
        pltpu.VMEM((W, 64), jnp.float32),
        pltpu.VMEM((W, ROW), jnp.float32),
        pltpu.VMEM((W,), jnp.float32),
        pltpu.VMEM((W,), jnp.float32),
        pltpu.VMEM((L,), jnp.float32),
        pltpu.VMEM_SHARED((N, 64), jnp.float32),
        pltpu.VMEM_SHARED((N, ROW), jnp.float32),
        pltpu.SemaphoreType.DMA,
        pltpu.SemaphoreType.DMA,
    ],
)(_sc_body)


def _finish_body(raw_ref, bias_ref, out_ref):
    r0 = raw_ref[0]
    r1 = raw_ref[1]
    eps = 1e-16
    parts = jnp.concatenate([
        r0[:, 0:32] / (r0[:, 64:65] + eps),
        r0[:, 32:64] / (r0[:, 65:66] + eps),
        r1[:, 0:32] / (r1[:, 64:65] + eps),
        r1[:, 32:64] / (r1[:, 65:66] + eps),
    ], axis=1)
    out_ref[...] = parts + bias_ref[...]


def _finish(raw, bias2d):
    return pl.pallas_call(
        _finish_body,
        grid=(GRID,),
        in_specs=[
            pl.BlockSpec((NC, HB, ROW), lambda i: (0, i, 0)),
            pl.BlockSpec((1, F), lambda i: (0, 0)),
        ],
        out_specs=pl.BlockSpec((HB, F), lambda i: (i, 0)),
        out_shape=jax.ShapeDtypeStruct((N, F), jnp.float32),
    )(raw, bias2d)


def kernel(x, edge_index, weight, att_weight, bias):
    w2d = weight.reshape(F, H * O)
    eye = jnp.eye(H, dtype=jnp.float32)
    a1 = att_weight[:, :O]
    a2 = att_weight[:, O:]
    amat1 = (eye[:, None, :] * a1[:, :, None]).reshape(F, H)
    amat2 = (eye[:, None, :] * a2[:, :, None]).reshape(F, H)
    amat = jnp.concatenate([amat1, amat2], axis=1)

    h, s, smax = _proj(x, w2d, amat)

    smax = smax[0]
    mh = jnp.maximum(smax[:H] + smax[H:], 0.0)
    mrow = jnp.zeros((NC, L), jnp.float32)
    mrow = mrow.at[0, 0:2].set(mh[0:2]).at[1, 0:2].set(mh[2:4])

    s1 = s[:, :H]
    s2 = s[:, H:]
    s1sc = jnp.stack([s1[:, 0:2].reshape(2 * N), s1[:, 2:4].reshape(2 * N)])
    s2sc = jnp.stack([s2[:, 0:2].reshape(2 * N), s2[:, 2:4].reshape(2 * N)])

    src_r = edge_index[0].astype(jnp.int32).reshape(NS, NWIN, W)
    dst_r = edge_index[1].astype(jnp.int32).reshape(NS, NWIN, W)

    raw = _sc_call(h, src_r, dst_r, s1sc, s2sc, mrow)

    bias2d = bias.reshape(1, F)
    return _finish(raw, bias2d)


# speedup vs baseline: 79.2126x; 79.2126x over previous
import functools

import jax
import jax.numpy as jnp
from jax import lax
from jax.experimental import pallas as pl
from jax.experimental.pallas import tpu as pltpu
from jax.experimental.pallas import tpu_sc as plsc

N = 10000
E = 320000
F = 128
H = 4
O = 32

NC = 2
NS = 16
L = 16

EPT = E // NS
W = 160
NWIN = EPT // W
NPT = N // NS
DW = 8
HB = 400
GRID = N // HB


def _proj_body(x_ref, w_ref, a_ref, h_ref, s_ref, smax_ref):
    i = pl.program_id(0)
    hb = jnp.dot(x_ref[...], w_ref[...], preferred_element_type=jnp.float32)
    h_ref[...] = hb
    sb = jnp.dot(hb, a_ref[...], preferred_element_type=jnp.float32)
    s_ref[...] = sb
    bm = jnp.max(sb, axis=0, keepdims=True)
    prev = jnp.where(i == 0, jnp.full_like(bm, -jnp.inf), smax_ref[...])
    smax_ref[...] = jnp.maximum(prev, bm)


def _proj(x, w2d, amat):
    return pl.pallas_call(
        _proj_body,
        grid=(GRID,),
        in_specs=[
            pl.BlockSpec((HB, F), lambda i: (i, 0)),
            pl.BlockSpec((F, F), lambda i: (0, 0)),
            pl.BlockSpec((F, 2 * H), lambda i: (0, 0)),
        ],
        out_specs=[
            pl.BlockSpec((HB, F), lambda i: (i, 0)),
            pl.BlockSpec((HB, 2 * H), lambda i: (i, 0)),
            pl.BlockSpec((1, 2 * H), lambda i: (0, 0)),
        ],
        out_shape=[
            jax.ShapeDtypeStruct((N, F), jnp.float32),
            jax.ShapeDtypeStruct((N, 2 * H), jnp.float32),
            jax.ShapeDtypeStruct((1, 2 * H), jnp.float32),
        ],
    )(x, w2d, amat)


def _sc_body(h_hbm, srcr_hbm, dstr_hbm, s1_hbm, s2_hbm, m_hbm,
             msg_hbm, den_hbm,
             srcb, dstb, s1_t, s2_t, rows64, rowsm, dbuf, aexp_a, aexp_b, mv,
             acc_sh, den_sh, sem_g, sem_s, sem_d):
    c = lax.axis_index("c")
    t = lax.axis_index("s")
    h_c = h_hbm.at[c]
    src_tile = srcr_hbm.at[t]
    dst_tile = dstr_hbm.at[t]

    pltpu.sync_copy(s1_hbm.at[c], s1_t)
    pltpu.sync_copy(s2_hbm.at[c], s2_t)
    pltpu.sync_copy(m_hbm.at[pl.ds(c * L, L)], mv)

    z = jnp.zeros((L,), jnp.float32)
    lane = lax.iota(jnp.int32, L)
    zrow = lax.shift_right_logical(lane, 3)
    zcol = lane & 7
    for r in range(W):
        for k in range(4):
            rowsm[r, pl.ds(k * L, L)] = z
    for r in range(W // 2):
        plsc.store_scatter(dbuf, [zrow + 2 * r, zcol], z)
    nfull = NPT // W
    rem = NPT - nfull * W
    for k in range(nfull):
        pltpu.sync_copy(rowsm, acc_sh.at[pl.ds(t * NPT + k * W, W)])
        pltpu.sync_copy(dbuf, den_sh.at[pl.ds(t * NPT + k * W, W)])
    pltpu.sync_copy(rowsm.at[pl.ds(0, rem)],
                    acc_sh.at[pl.ds(t * NPT + nfull * W, rem)])
    pltpu.sync_copy(dbuf.at[pl.ds(0, rem)],
                    den_sh.at[pl.ds(t * NPT + nfull * W, rem)])
    plsc.subcore_barrier()

    m = mv[...]
    ma = m[0]
    mb = m[1]
    col0 = jnp.full((L,), 0, jnp.int32)
    col1 = jnp.full((L,), 1, jnp.int32)

    def win_body(w, carry):
        pltpu.sync_copy(src_tile.at[w], srcb)
        pltpu.sync_copy(dst_tile.at[w], dstb)
        gcp = pltpu.async_copy(h_c.at[srcb], rows64, sem_g)
        for v in range(W // L):
            sv = srcb[pl.ds(v * L, L)]
            dv = dstb[pl.ds(v * L, L)]
            i1 = sv * 2
            i2 = dv * 2
            s1a = plsc.load_gather(s1_t, [i1])
            s1b = plsc.load_gather(s1_t, [i1 + 1])
            s2a = plsc.load_gather(s2_t, [i2])
            s2b = plsc.load_gather(s2_t, [i2 + 1])
            aa = s1a + s2a
            ab = s1b + s2b
            aa = jnp.where(aa > 0, aa, 0.2 * aa) - ma
            ab = jnp.where(ab > 0, ab, 0.2 * ab) - mb
            ea = jnp.exp(aa)
            eb = jnp.exp(ab)
            aexp_a[pl.ds(v * L, L)] = ea
            aexp_b[pl.ds(v * L, L)] = eb
            rowv = lane + (v * L)
            plsc.store_scatter(dbuf, [rowv, col0], ea)
            plsc.store_scatter(dbuf, [rowv, col1], eb)
        dcp = pltpu.async_copy(dbuf, den_sh.at[dstb], sem_d, add=True)
        gcp.wait()
        for v in range(W // L):
            ea = aexp_a[pl.ds(v * L, L)]
            eb = aexp_b[pl.ds(v * L, L)]
            for j in range(L):
                e = v * L + j
                sa = ea[j]
                sb = eb[j]
                rowsm[e, pl.ds(0, L)] = rows64[e, pl.ds(0, L)] * sa
                rowsm[e, pl.ds(L, L)] = rows64[e, pl.ds(L, L)] * sa
                rowsm[e, pl.ds(2 * L, L)] = rows64[e, pl.ds(2 * L, L)] * sb
                rowsm[e, pl.ds(3 * L, L)] = rows64[e, pl.ds(3 * L, L)] * sb
        pltpu.async_copy(rowsm, acc_sh.at[dstb], sem_s, add=True).wait()
        dcp.wait()
        return carry

    lax.fori_loop(0, NWIN, win_body, 0)
    plsc.subcore_barrier()
    pltpu.sync_copy(acc_sh.at[pl.ds(t * NPT, NPT)],
                    msg_hbm.at[c, pl.ds(t * NPT, NPT)])
    pltpu.sync_copy(den_sh.at[pl.ds(t * NPT, NPT)],
                    den_hbm.at[c, pl.ds(t * NPT, NPT)])


_sc_mesh = plsc.VectorSubcoreMesh(
    core_axis_name="c", subcore_axis_name="s", num_cores=NC, num_subcores=NS)

_sc_call = functools.partial(
    pl.kernel,
    out_type=(jax.ShapeDtypeStruct((NC, N, 64), jnp.float32),
              jax.ShapeDtypeStruct((NC, N, DW), jnp.float32)),
    mesh=_sc_mesh,
    compiler_params=pltpu.CompilerParams(
        needs_layout_passes=False, use_tc_tiling_on_sc=False),
    scratch_types=[
        pltpu.VMEM((W,), jnp.int32),
        pltpu.VMEM((W,), jnp.int32),
        pltpu.VMEM((2 * N,), jnp.float32),
        pltpu.VMEM((2 * N,), jnp.float32),
        pltpu.VMEM((W, 64), jnp.float32),
        pltpu.VMEM((W, 64), jnp.float32),
        pltpu.VMEM((W, DW), jnp.float32),
        pltpu.VMEM((W,), jnp.float32),
        pltpu.VMEM((W,), jnp.float32),
        pltpu.VMEM((L,), jnp.float32),
        pltpu.VMEM_SHARED((N, 64), jnp.float32),
        pltpu.VMEM_SHARED((N, DW), jnp.float32),
        pltpu.SemaphoreType.DMA,
        pltpu.SemaphoreType.DMA,
        pltpu.SemaphoreType.DMA,
    ],
)(_sc_body)


def _finish_body(msg_ref, den_ref, bias_ref, out_ref):
    m0 = msg_ref[0]
    m1 = msg_ref[1]
    d0 = den_ref[0]
    d1 = den_ref[1]
    eps = 1e-16
    parts = jnp.concatenate([
        m0[:, 0:32] / (d0[:, 0:1] + eps),
        m0[:, 32:64] / (d0[:, 1:2] + eps),
        m1[:, 0:32] / (d1[:, 0:1] + eps),
        m1[:, 32:64] / (d1[:, 1:2] + eps),
    ], axis=1)
    out_ref[...] = parts + bias_ref[...]


def _finish(msg, den, bias2d):
    return pl.pallas_call(
        _finish_body,
        grid=(GRID,),
        in_specs=[
            pl.BlockSpec((NC, HB, 64), lambda i: (0, i, 0)),
            pl.BlockSpec((NC, HB, DW), lambda i: (0, i, 0)),
            pl.BlockSpec((1, F), lambda i: (0, 0)),
        ],
        out_specs=pl.BlockSpec((HB, F), lambda i: (i, 0)),
        out_shape=jax.ShapeDtypeStruct((N, F), jnp.float32),
    )(msg, den, bias2d)


def kernel(x, edge_index, weight, att_weight, bias):
    w2d = weight.reshape(F, H * O)
    eye = jnp.eye(H, dtype=jnp.float32)
    a1 = att_weight[:, :O]
    a2 = att_weight[:, O:]
    amat1 = (eye[:, None, :] * a1[:, :, None]).reshape(F, H)
    amat2 = (eye[:, None, :] * a2[:, :, None]).reshape(F, H)
    amat = jnp.concatenate([amat1, amat2], axis=1)

    h, s, smax = _proj(x, w2d, amat)

    smax = smax[0]
    mh = jnp.maximum(smax[:H] + smax[H:], 0.0)
    mrow = jnp.zeros((NC * L,), jnp.float32)
    mrow = mrow.at[0:2].set(mh[0:2]).at[L:L + 2].set(mh[2:4])

    s1 = s[:, :H]
    s2 = s[:, H:]
    s1sc = jnp.stack([s1[:, 0:2].reshape(2 * N), s1[:, 2:4].reshape(2 * N)])
    s2sc = jnp.stack([s2[:, 0:2].reshape(2 * N), s2[:, 2:4].reshape(2 * N)])

    src_r = edge_index[0].astype(jnp.int32).reshape(NS, NWIN, W)
    dst_r = edge_index[1].astype(jnp.int32).reshape(NS, NWIN, W)

    h_sc = jnp.stack([h[:, :64], h[:, 64:]])

    msg, den = _sc_call(h_sc, src_r, dst_r, s1sc, s2sc, mrow)

    bias2d = bias.reshape(1, F)
    return _finish(msg, den, bias2d)
